# Initial kernel scaffold; baseline (speedup 1.0000x reference)
import jax
import jax.numpy as jnp
from jax import lax
from jax.experimental import pallas as pl
from jax.experimental.pallas import tpu as pltpu
from jax.experimental.pallas import tpu_sc as plsc

_VOCAB = 100000
_D = 128
_HID = 512
_B = 4096
_SEQ = 200

_NC = 2
_NS = 16
_NW = _NC * _NS
_BPW = _B // _NW
_CHUNK = 128
_NCHUNK = _BPW * _SEQ // _CHUNK
_NBUF = 4
_ACC_ROWS = _NS * _BPW


def _pool_body(idx_hbm, scat_hbm, table_hbm, out_hbm,
               idx_v, scat_v, buf0, buf1, buf2, buf3, acc,
               g0, g1, g2, g3, s0, s1, s2, s3):
    bufs = (buf0, buf1, buf2, buf3)
    gsems = (g0, g1, g2, g3)
    ssems = (s0, s1, s2, s3)
    c = lax.axis_index("c")
    s = lax.axis_index("s")
    wid = c * _NS + s

    pltpu.sync_copy(idx_hbm.at[wid], idx_v)
    pltpu.sync_copy(scat_hbm.at[s], scat_v)

    zero16 = jnp.zeros((16,), jnp.float32)

    @pl.loop(0, _CHUNK)
    def _(r):
        @pl.loop(0, _D, step=16)
        def _(k):
            buf0[r, pl.ds(k, 16)] = zero16

    pltpu.sync_copy(buf0, acc.at[pl.ds(s * _BPW, _BPW)])

    for b in range(_NBUF):
        pltpu.async_copy(table_hbm.at[idx_v.at[b]], bufs[b], gsems[b])

    @pl.loop(0, _NCHUNK, step=_NBUF)
    def _(j):
        for b in range(_NBUF):
            jj = j + b
            pltpu.make_async_copy(
                table_hbm.at[idx_v.at[jj]], bufs[b], gsems[b]).wait()
            pltpu.async_copy(bufs[b], acc.at[scat_v.at[jj]], ssems[b],
                             add=True)

            @pl.when(jj < _NCHUNK - _NBUF)
            def _():
                pltpu.make_async_copy(
                    bufs[b], acc.at[scat_v.at[jj]], ssems[b]).wait()
                pltpu.async_copy(
                    table_hbm.at[idx_v.at[jj + _NBUF]], bufs[b], gsems[b])

    for b in range(_NBUF):
        jj = _NCHUNK - _NBUF + b
        pltpu.make_async_copy(bufs[b], acc.at[scat_v.at[jj]],
                              ssems[b]).wait()
    pltpu.sync_copy(acc.at[pl.ds(s * _BPW, _BPW)],
                    out_hbm.at[pl.ds(wid * _BPW, _BPW)])


@jax.jit
def _sc_pool(idx3, scat_map, table):
    mesh = plsc.VectorSubcoreMesh(core_axis_name="c", subcore_axis_name="s")
    f = pl.kernel(
        _pool_body,
        out_type=jax.ShapeDtypeStruct((_B, _D), jnp.float32),
        mesh=mesh,
        scratch_types=[
            pltpu.VMEM((_NCHUNK, _CHUNK), jnp.int32),
            pltpu.VMEM((_NCHUNK, _CHUNK), jnp.int32),
            pltpu.VMEM((_CHUNK, _D), jnp.float32),
            pltpu.VMEM((_CHUNK, _D), jnp.float32),
            pltpu.VMEM((_CHUNK, _D), jnp.float32),
            pltpu.VMEM((_CHUNK, _D), jnp.float32),
            pltpu.VMEM_SHARED((_ACC_ROWS, _D), jnp.float32),
        ] + [pltpu.SemaphoreType.DMA] * 8,
    )
    return f(idx3, scat_map, table)


def _mlp_body(x_ref, w1_ref, b1_ref, w2_ref, b2_ref, o_ref):
    x = x_ref[...] * jnp.float32(1.0 / _SEQ)
    h = jnp.dot(x, w1_ref[...], preferred_element_type=jnp.float32)
    h = jnp.maximum(h + b1_ref[...], 0.0)
    z = jnp.dot(h, w2_ref[...], preferred_element_type=jnp.float32)
    z = z + b2_ref[...]
    m = jnp.max(z, axis=1, keepdims=True)
    e = jnp.exp(z - m)
    lse = jnp.log(jnp.sum(e, axis=1, keepdims=True)) + m
    o_ref[...] = z - lse


@jax.jit
def _tc_mlp(pooled, W1, b1, W2pad, b2pad):
    return pl.pallas_call(
        _mlp_body,
        out_shape=jax.ShapeDtypeStruct((_B, _D), jnp.float32),
    )(pooled, W1, b1, W2pad, b2pad)


@jax.jit
def kernel(sentence_indices, table, W1, b1, W2, b2):
    idx3 = sentence_indices.astype(jnp.int32).reshape(_NW, _NCHUNK, _CHUNK)
    rel = (jnp.arange(_BPW * _SEQ, dtype=jnp.int32) // _SEQ).reshape(
        _NCHUNK, _CHUNK)
    scat_map = rel[None] + (jnp.arange(_NS, dtype=jnp.int32) * _BPW)[
        :, None, None]
    pooled = _sc_pool(idx3, scat_map, table)

    W2pad = jnp.zeros((_HID, _D), jnp.float32).at[:, :2].set(W2)
    b2pad = jnp.full((1, _D), -1e30, jnp.float32).at[0, :2].set(b2)
    out = _tc_mlp(pooled, W1, b1.reshape(1, _HID), W2pad, b2pad)
    return out[:, :2]


# speedup vs baseline: 9.3523x; 9.3523x over previous
import jax
import jax.numpy as jnp
from jax import lax
from jax.experimental import pallas as pl
from jax.experimental.pallas import tpu as pltpu
from jax.experimental.pallas import tpu_sc as plsc

_VOCAB = 100000
_D = 128
_HID = 512
_B = 4096
_SEQ = 200

_NC = 2
_NS = 16
_NW = _NC * _NS
_BPW = _B // _NW
_CHUNK = 128
_NCHUNK = _BPW * _SEQ // _CHUNK
_NBUF = 2
_ACC_ROWS = _NS * _BPW


def _pool_body(idx_hbm, scat_hbm, table_hbm, out_hbm,
               idx_v, scat_v, buf0, buf1, acc,
               g0, g1, s0, s1):
    bufs = (buf0, buf1)
    gsems = (g0, g1)
    ssems = (s0, s1)
    c = lax.axis_index("c")
    s = lax.axis_index("s")
    wid = c * _NS + s

    pltpu.sync_copy(idx_hbm.at[wid], idx_v)
    pltpu.sync_copy(scat_hbm.at[s], scat_v)

    zero16 = jnp.zeros((16,), jnp.float32)

    @pl.loop(0, _CHUNK)
    def _(r):
        @pl.loop(0, _D, step=16)
        def _(k):
            buf0[r, pl.ds(k, 16)] = zero16

    pltpu.sync_copy(buf0, acc.at[pl.ds(s * _BPW, _BPW)])

    for b in range(_NBUF):
        pltpu.async_copy(table_hbm.at[idx_v.at[b]], bufs[b], gsems[b])

    @pl.loop(0, _NCHUNK, step=_NBUF)
    def _(j):
        for b in range(_NBUF):
            jj = j + b
            pltpu.make_async_copy(
                table_hbm.at[idx_v.at[jj]], bufs[b], gsems[b]).wait()
            pltpu.async_copy(bufs[b], acc.at[scat_v.at[jj]], ssems[b],
                             add=True)

            @pl.when(jj < _NCHUNK - _NBUF)
            def _():
                pltpu.make_async_copy(
                    bufs[b], acc.at[scat_v.at[jj]], ssems[b]).wait()
                pltpu.async_copy(
                    table_hbm.at[idx_v.at[jj + _NBUF]], bufs[b], gsems[b])

    for b in range(_NBUF):
        jj = _NCHUNK - _NBUF + b
        pltpu.make_async_copy(bufs[b], acc.at[scat_v.at[jj]],
                              ssems[b]).wait()
    pltpu.sync_copy(acc.at[pl.ds(s * _BPW, _BPW)],
                    out_hbm.at[pl.ds(wid * _BPW, _BPW)])


@jax.jit
def _sc_pool(idx3, scat_map, table):
    mesh = plsc.VectorSubcoreMesh(core_axis_name="c", subcore_axis_name="s")
    f = pl.kernel(
        _pool_body,
        out_type=jax.ShapeDtypeStruct((_B, _D), jnp.float32),
        mesh=mesh,
        scratch_types=[
            pltpu.VMEM((_NCHUNK, _CHUNK), jnp.int32),
            pltpu.VMEM((_NCHUNK, _CHUNK), jnp.int32),
            pltpu.VMEM((_CHUNK, _D), jnp.float32),
            pltpu.VMEM((_CHUNK, _D), jnp.float32),
            pltpu.VMEM_SHARED((_ACC_ROWS, _D), jnp.float32),
        ] + [pltpu.SemaphoreType.DMA] * 4,
    )
    return f(idx3, scat_map, table)


def _mlp_body(x_ref, w1_ref, b1_ref, w2_ref, b2_ref, o_ref):
    x = x_ref[...] * jnp.float32(1.0 / _SEQ)
    h = jnp.dot(x, w1_ref[...], preferred_element_type=jnp.float32)
    h = jnp.maximum(h + b1_ref[...], 0.0)
    z = jnp.dot(h, w2_ref[...], preferred_element_type=jnp.float32)
    z = z + b2_ref[...]
    m = jnp.max(z, axis=1, keepdims=True)
    e = jnp.exp(z - m)
    lse = jnp.log(jnp.sum(e, axis=1, keepdims=True)) + m
    o_ref[...] = z - lse


@jax.jit
def _tc_mlp(pooled, W1, b1, W2pad, b2pad):
    return pl.pallas_call(
        _mlp_body,
        out_shape=jax.ShapeDtypeStruct((_B, _D), jnp.float32),
    )(pooled, W1, b1, W2pad, b2pad)


@jax.jit
def kernel(sentence_indices, table, W1, b1, W2, b2):
    idx3 = sentence_indices.astype(jnp.int32).reshape(_NW, _NCHUNK, _CHUNK)
    rel = (jnp.arange(_BPW * _SEQ, dtype=jnp.int32) // _SEQ).reshape(
        _NCHUNK, _CHUNK)
    scat_map = rel[None] + (jnp.arange(_NS, dtype=jnp.int32) * _BPW)[
        :, None, None]
    pooled = _sc_pool(idx3, scat_map, table)

    W2pad = jnp.zeros((_HID, _D), jnp.float32).at[:, :2].set(W2)
    b2pad = jnp.full((1, _D), -1e30, jnp.float32).at[0, :2].set(b2)
    out = _tc_mlp(pooled, W1, b1.reshape(1, _HID), W2pad, b2pad)
    return out[:, :2]


# speedup vs baseline: 9.4479x; 1.0102x over previous
import jax
import jax.numpy as jnp
from jax import lax
from jax.experimental import pallas as pl
from jax.experimental.pallas import tpu as pltpu
from jax.experimental.pallas import tpu_sc as plsc

_VOCAB = 100000
_D = 128
_HID = 512
_B = 4096
_SEQ = 200

_NC = 2
_NS = 16
_NW = _NC * _NS
_BPW = _B // _NW
_CHUNK = 128
_NCHUNK = _BPW * _SEQ // _CHUNK
_NBUF = 3
_ACC_ROWS = _NS * _BPW


def _pool_body(idx_hbm, scat_hbm, table_hbm, out_hbm,
               idx_v, scat_v, buf0, buf1, buf2, acc,
               g0, g1, g2, s0, s1, s2):
    bufs = (buf0, buf1, buf2)
    gsems = (g0, g1, g2)
    ssems = (s0, s1, s2)
    c = lax.axis_index("c")
    s = lax.axis_index("s")
    wid = c * _NS + s

    pltpu.sync_copy(idx_hbm.at[wid], idx_v)
    pltpu.sync_copy(scat_hbm.at[s], scat_v)

    zero16 = jnp.zeros((16,), jnp.float32)

    @pl.loop(0, _BPW)
    def _(r):
        @pl.loop(0, _D, step=16)
        def _(k):
            buf0[r, pl.ds(k, 16)] = zero16

    pltpu.sync_copy(buf0, acc.at[pl.ds(s * _BPW, _BPW)])

    for b in range(_NBUF):
        pltpu.async_copy(table_hbm.at[idx_v.at[b]], bufs[b], gsems[b])

    _MAIN = _NCHUNK - (_NCHUNK % _NBUF)

    def slot(jj, b, last_gather):
        pltpu.make_async_copy(
            table_hbm.at[idx_v.at[jj]], bufs[b], gsems[b]).wait()
        pltpu.async_copy(bufs[b], acc.at[scat_v.at[jj]], ssems[b],
                         add=True)

        @pl.when(jj < last_gather)
        def _():
            pltpu.make_async_copy(
                bufs[b], acc.at[scat_v.at[jj]], ssems[b]).wait()
            pltpu.async_copy(
                table_hbm.at[idx_v.at[jj + _NBUF]], bufs[b], gsems[b])

    @pl.loop(0, _MAIN, step=_NBUF)
    def _(j):
        for b in range(_NBUF):
            slot(j + b, b, _NCHUNK - _NBUF)

    for jj in range(_MAIN, _NCHUNK):
        slot(jj, jj % _NBUF, _MAIN - _NBUF)

    for jj in range(_NCHUNK - _NBUF, _NCHUNK):
        pltpu.make_async_copy(bufs[jj % _NBUF], acc.at[scat_v.at[jj]],
                              ssems[jj % _NBUF]).wait()
    pltpu.sync_copy(acc.at[pl.ds(s * _BPW, _BPW)],
                    out_hbm.at[pl.ds(wid * _BPW, _BPW)])


@jax.jit
def _sc_pool(idx3, scat_map, table):
    mesh = plsc.VectorSubcoreMesh(core_axis_name="c", subcore_axis_name="s")
    f = pl.kernel(
        _pool_body,
        out_type=jax.ShapeDtypeStruct((_B, _D), jnp.float32),
        mesh=mesh,
        scratch_types=[
            pltpu.VMEM((_NCHUNK, _CHUNK), jnp.int32),
            pltpu.VMEM((_NCHUNK, _CHUNK), jnp.int32),
            pltpu.VMEM((_CHUNK, _D), jnp.float32),
            pltpu.VMEM((_CHUNK, _D), jnp.float32),
            pltpu.VMEM((_CHUNK, _D), jnp.float32),
            pltpu.VMEM_SHARED((_ACC_ROWS, _D), jnp.float32),
        ] + [pltpu.SemaphoreType.DMA] * 6,
    )
    return f(idx3, scat_map, table)


def _mlp_body(x_ref, w1_ref, b1_ref, w2_ref, b2_ref, o_ref):
    x = x_ref[...] * jnp.float32(1.0 / _SEQ)
    h = jnp.dot(x, w1_ref[...], preferred_element_type=jnp.float32)
    h = jnp.maximum(h + b1_ref[...], 0.0)
    z = jnp.dot(h, w2_ref[...], preferred_element_type=jnp.float32)
    z = z + b2_ref[...]
    m = jnp.max(z, axis=1, keepdims=True)
    e = jnp.exp(z - m)
    lse = jnp.log(jnp.sum(e, axis=1, keepdims=True)) + m
    o_ref[...] = z - lse


@jax.jit
def _tc_mlp(pooled, W1, b1, W2pad, b2pad):
    return pl.pallas_call(
        _mlp_body,
        out_shape=jax.ShapeDtypeStruct((_B, _D), jnp.float32),
    )(pooled, W1, b1, W2pad, b2pad)


@jax.jit
def kernel(sentence_indices, table, W1, b1, W2, b2):
    idx3 = sentence_indices.astype(jnp.int32).reshape(_NW, _NCHUNK, _CHUNK)
    rel = (jnp.arange(_BPW * _SEQ, dtype=jnp.int32) // _SEQ).reshape(
        _NCHUNK, _CHUNK)
    scat_map = rel[None] + (jnp.arange(_NS, dtype=jnp.int32) * _BPW)[
        :, None, None]
    pooled = _sc_pool(idx3, scat_map, table)

    W2pad = jnp.zeros((_HID, _D), jnp.float32).at[:, :2].set(W2)
    b2pad = jnp.full((1, _D), -1e30, jnp.float32).at[0, :2].set(b2)
    out = _tc_mlp(pooled, W1, b1.reshape(1, _HID), W2pad, b2pad)
    return out[:, :2]


# speedup vs baseline: 9.7788x; 1.0350x over previous
import jax
import jax.numpy as jnp
from jax import lax
from jax.experimental import pallas as pl
from jax.experimental.pallas import tpu as pltpu
from jax.experimental.pallas import tpu_sc as plsc

_VOCAB = 100000
_D = 128
_HID = 512
_B = 4096
_SEQ = 200

_NC = 2
_NS = 16
_NW = _NC * _NS
_BPW = _B // _NW
_CHUNK = 128
_NCHUNK = _BPW * _SEQ // _CHUNK
_NBUF = 5
_K = 2
_SCAT_PERIOD = 25
_ACC_ROWS = _NS * _BPW


def _pool_body(idx_hbm, scat_hbm, table_hbm, out_hbm,
               idx_v, scat_v, buf0, buf1, buf2, buf3, buf4,
               stg0, stg1, stg2, stg3, stg4, acc,
               g0, g1, g2, g3, g4, s0, s1, s2, s3, s4):
    bufs = (buf0, buf1, buf2, buf3, buf4)
    stgs = (stg0, stg1, stg2, stg3, stg4)
    gsems = (g0, g1, g2, g3, g4)
    ssems = (s0, s1, s2, s3, s4)
    c = lax.axis_index("c")
    s = lax.axis_index("s")
    wid = c * _NS + s

    pltpu.sync_copy(idx_hbm.at[wid], idx_v)
    pltpu.sync_copy(scat_hbm, scat_v)

    zero16 = jnp.zeros((16,), jnp.float32)

    @pl.loop(0, _BPW)
    def _(r):
        @pl.loop(0, _D, step=16)
        def _(k):
            buf0[r, pl.ds(k, 16)] = zero16

    pltpu.sync_copy(buf0, acc.at[pl.ds(s * _BPW, _BPW)])

    for b in range(_NBUF - _K):
        pltpu.async_copy(table_hbm.at[idx_v.at[b]], bufs[b], gsems[b])

    def slot(jj, b):
        b2 = (b + _NBUF - _K) % _NBUF
        pltpu.make_async_copy(
            table_hbm.at[idx_v.at[jj]], bufs[b], gsems[b]).wait()
        off = (jj // _SCAT_PERIOD) * 16 + s * _BPW
        r = jj % _SCAT_PERIOD
        for m in range(_CHUNK // 16):
            stgs[b][pl.ds(16 * m, 16)] = scat_v[r, pl.ds(16 * m, 16)] + off
        pltpu.async_copy(bufs[b], acc.at[stgs[b]], ssems[b], add=True)

        nxt = jj + _NBUF - _K
        @pl.when(nxt < _NCHUNK)
        def _():
            @pl.when(jj >= _K)
            def _():
                pltpu.make_async_copy(
                    bufs[b2], acc.at[stgs[b2]], ssems[b2]).wait()
            pltpu.async_copy(
                table_hbm.at[idx_v.at[nxt]], bufs[b2], gsems[b2])

    @pl.loop(0, _NCHUNK, step=_NBUF)
    def _(j):
        for b in range(_NBUF):
            slot(j + b, b)

    for jj in range(_NCHUNK - _NBUF, _NCHUNK):
        bd = jj % _NBUF
        pltpu.make_async_copy(bufs[bd], acc.at[stgs[bd]],
                              ssems[bd]).wait()
    pltpu.sync_copy(acc.at[pl.ds(s * _BPW, _BPW)],
                    out_hbm.at[pl.ds(wid * _BPW, _BPW)])


@jax.jit
def _sc_pool(idx3, scat_map, table):
    mesh = plsc.VectorSubcoreMesh(core_axis_name="c", subcore_axis_name="s")
    f = pl.kernel(
        _pool_body,
        out_type=jax.ShapeDtypeStruct((_B, _D), jnp.float32),
        mesh=mesh,
        scratch_types=[
            pltpu.VMEM((_NCHUNK, _CHUNK), jnp.int32),
            pltpu.VMEM((_SCAT_PERIOD, _CHUNK), jnp.int32),
        ] + [pltpu.VMEM((_CHUNK, _D), jnp.float32)] * _NBUF
          + [pltpu.VMEM((_CHUNK,), jnp.int32)] * _NBUF
          + [pltpu.VMEM_SHARED((_ACC_ROWS, _D), jnp.float32)]
          + [pltpu.SemaphoreType.DMA] * (2 * _NBUF),
    )
    return f(idx3, scat_map, table)


def _mlp_body(x_ref, w1_ref, b1_ref, w2_ref, b2_ref, o_ref):
    x = x_ref[...] * jnp.float32(1.0 / _SEQ)
    h = jnp.dot(x, w1_ref[...], preferred_element_type=jnp.float32)
    h = jnp.maximum(h + b1_ref[...], 0.0)
    z = jnp.dot(h, w2_ref[...], preferred_element_type=jnp.float32)
    z = z + b2_ref[...]
    m = jnp.max(z, axis=1, keepdims=True)
    e = jnp.exp(z - m)
    lse = jnp.log(jnp.sum(e, axis=1, keepdims=True)) + m
    o_ref[...] = z - lse


@jax.jit
def _tc_mlp(pooled, W1, b1, W2pad, b2pad):
    return pl.pallas_call(
        _mlp_body,
        out_shape=jax.ShapeDtypeStruct((_B, _D), jnp.float32),
    )(pooled, W1, b1, W2pad, b2pad)


@jax.jit
def kernel(sentence_indices, table, W1, b1, W2, b2):
    idx3 = sentence_indices.astype(jnp.int32).reshape(_NW, _NCHUNK, _CHUNK)
    rel = (jnp.arange(_SCAT_PERIOD * _CHUNK, dtype=jnp.int32)
           // _SEQ).reshape(_SCAT_PERIOD, _CHUNK)
    pooled = _sc_pool(idx3, rel, table)

    W2pad = jnp.zeros((_HID, _D), jnp.float32).at[:, :2].set(W2)
    b2pad = jnp.full((1, _D), -1e30, jnp.float32).at[0, :2].set(b2)
    out = _tc_mlp(pooled, W1, b1.reshape(1, _HID), W2pad, b2pad)
    return out[:, :2]
